# baseline (speedup 1.0000x reference)
import functools

import jax
import jax.numpy as jnp
from jax import lax
from jax.experimental import pallas as pl
from jax.experimental.pallas import tpu as pltpu
from jax.experimental.pallas import tpu_sc as plsc

T = 2048
D = 1024
H = 16
DH = 64
E = 8
F = 256
EPS = 1e-6
HD = H * DH

BT = 256
BQ = 512


def _rms(x, w):
    var = jnp.mean(x * x, axis=-1, keepdims=True)
    return x * lax.rsqrt(var + EPS) * w


def _embed_gather(table, idx):
    info = plsc.get_sparse_core_info()
    nw = info.num_cores * info.num_subcores
    b_per_w = T // nw
    mesh = plsc.VectorSubcoreMesh(core_axis_name="c", subcore_axis_name="s")

    @functools.partial(
        pl.kernel,
        mesh=mesh,
        out_type=jax.ShapeDtypeStruct((T, D), jnp.float32),
        scratch_types=[
            pltpu.VMEM((b_per_w,), jnp.int32),
            pltpu.VMEM((b_per_w, D), jnp.float32),
            pltpu.SemaphoreType.DMA,
        ],
    )
    def gather_k(table_hbm, idx_hbm, out_hbm, idx_v, rows_v, sem):
        wid = lax.axis_index("s") * info.num_cores + lax.axis_index("c")
        base = wid * b_per_w
        pltpu.sync_copy(idx_hbm.at[pl.ds(base, b_per_w)], idx_v)
        pltpu.async_copy(table_hbm.at[idx_v], rows_v, sem).wait()
        pltpu.sync_copy(rows_v, out_hbm.at[pl.ds(base, b_per_w)])

    return gather_k(table, idx)


def _prelude_body(emb_ref, spec_ref, ehp_ref, wq_ref, wqp_ref, wk_ref,
                  wkp_ref, wv_ref, cos_ref, sin_ref,
                  enw_ref, hnw_ref, ln1_ref,
                  res_ref, q_ref, k_ref, v_ref):
    en = _rms(emb_ref[...], enw_ref[...]).astype(jnp.bfloat16)
    hn = _rms(spec_ref[...], hnw_ref[...]).astype(jnp.bfloat16)
    x = (jnp.dot(en, ehp_ref[:D, :], preferred_element_type=jnp.float32)
         + jnp.dot(hn, ehp_ref[D:, :], preferred_element_type=jnp.float32))
    res_ref[...] = x
    hs = _rms(x, ln1_ref[...]).astype(jnp.bfloat16)
    cos, sin = cos_ref[...], sin_ref[...]
    q_ref[...] = (jnp.dot(hs, wq_ref[...], preferred_element_type=jnp.float32) * cos
                  + jnp.dot(hs, wqp_ref[...], preferred_element_type=jnp.float32) * sin
                  ).astype(jnp.bfloat16)
    k_ref[...] = (jnp.dot(hs, wk_ref[...], preferred_element_type=jnp.float32) * cos
                  + jnp.dot(hs, wkp_ref[...], preferred_element_type=jnp.float32) * sin
                  ).astype(jnp.bfloat16)
    v_ref[...] = jnp.dot(hs, wv_ref[...],
                         preferred_element_type=jnp.float32).astype(jnp.bfloat16)


def _prelude(emb, spec, ehp, wq, wqp, wk, wkp, wv, cos_t, sin_t, enw, hnw, ln1):
    grid = (T // BT,)
    tok = pl.BlockSpec((BT, D), lambda i: (i, 0))
    full = lambda shape: pl.BlockSpec(shape, lambda i: (0,) * len(shape))
    return pl.pallas_call(
        _prelude_body,
        grid=grid,
        in_specs=[tok, tok, full((2 * D, D)), full((D, HD)), full((D, HD)),
                  full((D, HD)), full((D, HD)), full((D, HD)),
                  tok, tok, full((1, D)), full((1, D)), full((1, D))],
        out_specs=[tok, pl.BlockSpec((BT, HD), lambda i: (i, 0)),
                   pl.BlockSpec((BT, HD), lambda i: (i, 0)),
                   pl.BlockSpec((BT, HD), lambda i: (i, 0))],
        out_shape=[jax.ShapeDtypeStruct((T, D), jnp.float32),
                   jax.ShapeDtypeStruct((T, HD), jnp.bfloat16),
                   jax.ShapeDtypeStruct((T, HD), jnp.bfloat16),
                   jax.ShapeDtypeStruct((T, HD), jnp.bfloat16)],
    )(emb, spec, ehp, wq, wqp, wk, wkp, wv, cos_t, sin_t, enw, hnw, ln1)


BK = 512


def _attn_body(q_ref, k_ref, v_ref, o_ref):
    iq = pl.program_id(1)
    rows = iq * BQ + lax.broadcasted_iota(jnp.int32, (BQ, BK), 0)
    cols0 = lax.broadcasted_iota(jnp.int32, (BQ, BK), 1)
    outs = []
    for hh in range(2):
        q = q_ref[:, hh * DH:(hh + 1) * DH]

        def body(j, carry):
            m, l, acc = carry
            kc = k_ref[pl.ds(j * BK, BK), hh * DH:(hh + 1) * DH]
            vc = v_ref[pl.ds(j * BK, BK), hh * DH:(hh + 1) * DH]
            s = lax.dot_general(q, kc, (((1,), (1,)), ((), ())),
                                preferred_element_type=jnp.float32) * 0.125
            s = jnp.where(j * BK + cols0 <= rows, s, -1e30)
            m_new = jnp.maximum(m, jnp.max(s, axis=-1, keepdims=True))
            alpha = jnp.exp(m - m_new)
            p = jnp.exp(s - m_new)
            l_new = l * alpha + jnp.sum(p, axis=-1, keepdims=True)
            acc_new = acc * alpha + jnp.dot(p.astype(jnp.bfloat16), vc,
                                            preferred_element_type=jnp.float32)
            return m_new, l_new, acc_new

        init = (jnp.full((BQ, 1), -1e30, jnp.float32),
                jnp.zeros((BQ, 1), jnp.float32),
                jnp.zeros((BQ, DH), jnp.float32))
        m, l, acc = lax.fori_loop(0, iq + 1, body, init)
        outs.append(acc / l)
    o_ref[...] = jnp.concatenate(outs, axis=1).astype(jnp.bfloat16)


def _attention(q, k, v):
    grid = (H // 2, T // BQ)
    return pl.pallas_call(
        _attn_body,
        grid=grid,
        in_specs=[
            pl.BlockSpec((BQ, 2 * DH), lambda h, i: (i, h)),
            pl.BlockSpec((T, 2 * DH), lambda h, i: (0, h)),
            pl.BlockSpec((T, 2 * DH), lambda h, i: (0, h)),
        ],
        out_specs=pl.BlockSpec((BQ, 2 * DH), lambda h, i: (i, h)),
        out_shape=jax.ShapeDtypeStruct((T, HD), jnp.bfloat16),
    )(q, k, v)


def _post_body(o_ref, x_ref, wo_ref, ln2_ref, rw_ref, wg_ref, wu_ref, wd_ref,
               fln_ref, out_ref):
    attn = jnp.dot(o_ref[...], wo_ref[...], preferred_element_type=jnp.float32)
    resid = x_ref[...] + attn
    hs = _rms(resid, ln2_ref[...])
    hsb = hs.astype(jnp.bfloat16)
    logits = jnp.dot(hs, rw_ref[...], preferred_element_type=jnp.float32)
    m = jnp.max(logits, axis=-1, keepdims=True)
    ex = jnp.exp(logits - m)
    probs = ex / jnp.sum(ex, axis=-1, keepdims=True)
    ii = lax.broadcasted_iota(jnp.int32, (BT, E), 1)
    t1 = jnp.max(probs, axis=-1, keepdims=True)
    a1 = jnp.min(jnp.where(probs == t1, ii, E), axis=-1, keepdims=True)
    sel1 = ii == a1
    p2 = jnp.where(sel1, -1.0, probs)
    t2 = jnp.max(p2, axis=-1, keepdims=True)
    a2 = jnp.min(jnp.where(p2 == t2, ii, E), axis=-1, keepdims=True)
    sel2 = ii == a2
    we = (jnp.where(sel1, t1, 0.0) + jnp.where(sel2, t2, 0.0)) / (t1 + t2)
    acc = jnp.zeros((BT, D), jnp.float32)
    for e in range(E):
        g = jnp.dot(hsb, wg_ref[e], preferred_element_type=jnp.float32)
        u = jnp.dot(hsb, wu_ref[e], preferred_element_type=jnp.float32)
        act = (g / (1.0 + jnp.exp(-g)) * u).astype(jnp.bfloat16)
        acc = acc + we[:, e:e + 1] * jnp.dot(
            act, wd_ref[e], preferred_element_type=jnp.float32)
    out_ref[...] = _rms(resid + acc, fln_ref[...])


def _post(o, x, wo, ln2, rw, wg, wu, wd, fln):
    grid = (T // BT,)
    tokd = pl.BlockSpec((BT, D), lambda i: (i, 0))
    full = lambda shape: pl.BlockSpec(shape, lambda i: (0,) * len(shape))
    return pl.pallas_call(
        _post_body,
        grid=grid,
        in_specs=[pl.BlockSpec((BT, HD), lambda i: (i, 0)), tokd,
                  full((HD, D)), full((1, D)), full((D, E)),
                  full((E, D, F)), full((E, D, F)), full((E, F, D)),
                  full((1, D))],
        out_specs=tokd,
        out_shape=jax.ShapeDtypeStruct((T, D), jnp.float32),
    )(o, x, wo, ln2, rw, wg, wu, wd, fln)


def kernel(input_ids, positions, spec_hidden, emb_table, enorm_w, hnorm_w,
           eh_proj_w, ln1_w, wq, wk, wv, wo, ln2_w, router_w, w_gate, w_up,
           w_down, final_ln_w):
    ids = input_ids.astype(jnp.int32)
    emb = _embed_gather(emb_table, ids)

    inv = 1.0 / (10000.0 ** (jnp.arange(0, DH, 2, dtype=jnp.float32) / DH))
    ang = positions.astype(jnp.float32)[:, None] * inv[None, :]
    cos_t = jnp.tile(jnp.cos(ang), (1, 2 * H))
    sin_t = jnp.tile(jnp.sin(ang), (1, 2 * H))

    def perm(w):
        w4 = w.reshape(D, H, 2, DH // 2)
        return jnp.concatenate([-w4[:, :, 1], w4[:, :, 0]], axis=2).reshape(D, HD)

    bf = lambda w: w.astype(jnp.bfloat16)
    res, q, k, v = _prelude(emb, spec_hidden, bf(eh_proj_w), bf(wq),
                            bf(perm(wq)), bf(wk), bf(perm(wk)), bf(wv),
                            cos_t, sin_t,
                            enorm_w.reshape(1, D), hnorm_w.reshape(1, D),
                            ln1_w.reshape(1, D))
    o = _attention(q, k, v)
    return _post(o, res, bf(wo), ln2_w.reshape(1, D), router_w, bf(w_gate),
                 bf(w_up), bf(w_down), final_ln_w.reshape(1, D))


# speedup vs baseline: 1.0678x; 1.0678x over previous
import functools

import jax
import jax.numpy as jnp
from jax import lax
from jax.experimental import pallas as pl
from jax.experimental.pallas import tpu as pltpu
from jax.experimental.pallas import tpu_sc as plsc

T = 2048
D = 1024
H = 16
DH = 64
E = 8
F = 256
EPS = 1e-6
HD = H * DH

BT = 256
BQ = 512


def _rms(x, w):
    var = jnp.mean(x * x, axis=-1, keepdims=True)
    return x * lax.rsqrt(var + EPS) * w


def _embed_gather(table, idx):
    info = plsc.get_sparse_core_info()
    nw = info.num_cores * info.num_subcores
    b_per_w = T // nw
    mesh = plsc.VectorSubcoreMesh(core_axis_name="c", subcore_axis_name="s")

    @functools.partial(
        pl.kernel,
        mesh=mesh,
        out_type=jax.ShapeDtypeStruct((T, D), jnp.float32),
        scratch_types=[
            pltpu.VMEM((b_per_w,), jnp.int32),
            pltpu.VMEM((b_per_w, D), jnp.float32),
            pltpu.SemaphoreType.DMA,
        ],
    )
    def gather_k(table_hbm, idx_hbm, out_hbm, idx_v, rows_v, sem):
        wid = lax.axis_index("s") * info.num_cores + lax.axis_index("c")
        base = wid * b_per_w
        pltpu.sync_copy(idx_hbm.at[pl.ds(base, b_per_w)], idx_v)
        pltpu.async_copy(table_hbm.at[idx_v], rows_v, sem).wait()
        pltpu.sync_copy(rows_v, out_hbm.at[pl.ds(base, b_per_w)])

    return gather_k(table, idx)


def _prelude_body(emb_ref, spec_ref, ehp_ref, wq_ref, wqp_ref, wk_ref,
                  wkp_ref, wv_ref, cos_ref, sin_ref,
                  enw_ref, hnw_ref, ln1_ref,
                  res_ref, q_ref, k_ref, v_ref):
    en = _rms(emb_ref[...], enw_ref[...])
    hn = _rms(spec_ref[...], hnw_ref[...])
    x = (jnp.dot(en, ehp_ref[:D, :], preferred_element_type=jnp.float32)
         + jnp.dot(hn, ehp_ref[D:, :], preferred_element_type=jnp.float32))
    res_ref[...] = x
    hs = _rms(x, ln1_ref[...])
    cos, sin = cos_ref[...], sin_ref[...]
    q_ref[...] = (jnp.dot(hs, wq_ref[...], preferred_element_type=jnp.float32) * cos
                  + jnp.dot(hs, wqp_ref[...], preferred_element_type=jnp.float32) * sin)
    k_ref[...] = (jnp.dot(hs, wk_ref[...], preferred_element_type=jnp.float32) * cos
                  + jnp.dot(hs, wkp_ref[...], preferred_element_type=jnp.float32) * sin)
    v_ref[...] = jnp.dot(hs, wv_ref[...], preferred_element_type=jnp.float32)


def _prelude(emb, spec, ehp, wq, wqp, wk, wkp, wv, cos_t, sin_t, enw, hnw, ln1):
    grid = (T // BT,)
    tok = pl.BlockSpec((BT, D), lambda i: (i, 0))
    full = lambda shape: pl.BlockSpec(shape, lambda i: (0,) * len(shape))
    return pl.pallas_call(
        _prelude_body,
        grid=grid,
        in_specs=[tok, tok, full((2 * D, D)), full((D, HD)), full((D, HD)),
                  full((D, HD)), full((D, HD)), full((D, HD)),
                  tok, tok, full((1, D)), full((1, D)), full((1, D))],
        out_specs=[tok, pl.BlockSpec((BT, HD), lambda i: (i, 0)),
                   pl.BlockSpec((BT, HD), lambda i: (i, 0)),
                   pl.BlockSpec((BT, HD), lambda i: (i, 0))],
        out_shape=[jax.ShapeDtypeStruct((T, D), jnp.float32),
                   jax.ShapeDtypeStruct((T, HD), jnp.float32),
                   jax.ShapeDtypeStruct((T, HD), jnp.float32),
                   jax.ShapeDtypeStruct((T, HD), jnp.float32)],
    )(emb, spec, ehp, wq, wqp, wk, wkp, wv, cos_t, sin_t, enw, hnw, ln1)


BK = 512


def _attn_body(q_ref, k_ref, v_ref, o_ref):
    iq = pl.program_id(1)
    rkey = lax.broadcasted_iota(jnp.int32, (BK, BQ), 0)
    cquery = iq * BQ + lax.broadcasted_iota(jnp.int32, (BK, BQ), 1)
    outs = []
    for hh in range(2):
        q = q_ref[:, hh * DH:(hh + 1) * DH]

        def body(j, carry):
            m, l, acc_t = carry
            kc = k_ref[pl.ds(j * BK, BK), hh * DH:(hh + 1) * DH]
            vc = v_ref[pl.ds(j * BK, BK), hh * DH:(hh + 1) * DH]
            st = lax.dot_general(kc, q, (((1,), (1,)), ((), ())),
                                 preferred_element_type=jnp.float32) * 0.125
            st = jnp.where(j * BK + rkey <= cquery, st, -1e30)
            m_new = jnp.maximum(m, jnp.max(st, axis=0, keepdims=True))
            alpha = jnp.exp(m - m_new)
            p = jnp.exp(st - m_new)
            l_new = l * alpha + jnp.sum(p, axis=0, keepdims=True)
            acc_t_new = acc_t * alpha + lax.dot_general(
                vc, p, (((0,), (0,)), ((), ())),
                preferred_element_type=jnp.float32)
            return m_new, l_new, acc_t_new

        init = (jnp.full((1, BQ), -1e30, jnp.float32),
                jnp.zeros((1, BQ), jnp.float32),
                jnp.zeros((DH, BQ), jnp.float32))
        m, l, acc_t = lax.fori_loop(0, iq + 1, body, init)
        outs.append(acc_t / l)
    o_ref[...] = jnp.concatenate(outs, axis=0)


def _attention(q, k, v):
    grid = (H // 2, T // BQ)
    return pl.pallas_call(
        _attn_body,
        grid=grid,
        in_specs=[
            pl.BlockSpec((BQ, 2 * DH), lambda h, i: (i, h)),
            pl.BlockSpec((T, 2 * DH), lambda h, i: (0, h)),
            pl.BlockSpec((T, 2 * DH), lambda h, i: (0, h)),
        ],
        out_specs=pl.BlockSpec((2 * DH, BQ), lambda h, i: (h, i)),
        out_shape=jax.ShapeDtypeStruct((HD, T), jnp.float32),
    )(q, k, v)


def _post_body(o_ref, x_ref, wo_ref, ln2_ref, rw_ref, wg_ref, wu_ref, wd_ref,
               fln_ref, out_ref):
    attn = lax.dot_general(o_ref[...], wo_ref[...], (((0,), (0,)), ((), ())),
                           preferred_element_type=jnp.float32)
    resid = x_ref[...] + attn
    hs = _rms(resid, ln2_ref[...])
    logits = jnp.dot(hs, rw_ref[...], preferred_element_type=jnp.float32)
    m = jnp.max(logits, axis=-1, keepdims=True)
    ex = jnp.exp(logits - m)
    probs = ex / jnp.sum(ex, axis=-1, keepdims=True)
    ii = lax.broadcasted_iota(jnp.int32, (BT, E), 1)
    t1 = jnp.max(probs, axis=-1, keepdims=True)
    a1 = jnp.min(jnp.where(probs == t1, ii, E), axis=-1, keepdims=True)
    sel1 = ii == a1
    p2 = jnp.where(sel1, -1.0, probs)
    t2 = jnp.max(p2, axis=-1, keepdims=True)
    a2 = jnp.min(jnp.where(p2 == t2, ii, E), axis=-1, keepdims=True)
    sel2 = ii == a2
    we = (jnp.where(sel1, t1, 0.0) + jnp.where(sel2, t2, 0.0)) / (t1 + t2)
    acc = jnp.zeros((BT, D), jnp.float32)
    for e in range(E):
        g = jnp.dot(hs, wg_ref[e], preferred_element_type=jnp.float32)
        u = jnp.dot(hs, wu_ref[e], preferred_element_type=jnp.float32)
        act = g / (1.0 + jnp.exp(-g)) * u
        acc = acc + we[:, e:e + 1] * jnp.dot(
            act, wd_ref[e], preferred_element_type=jnp.float32)
    out_ref[...] = _rms(resid + acc, fln_ref[...])


def _post(o, x, wo, ln2, rw, wg, wu, wd, fln):
    grid = (T // BT,)
    tokd = pl.BlockSpec((BT, D), lambda i: (i, 0))
    full = lambda shape: pl.BlockSpec(shape, lambda i: (0,) * len(shape))
    return pl.pallas_call(
        _post_body,
        grid=grid,
        in_specs=[pl.BlockSpec((HD, BT), lambda i: (0, i)), tokd,
                  full((HD, D)), full((1, D)), full((D, E)),
                  full((E, D, F)), full((E, D, F)), full((E, F, D)),
                  full((1, D))],
        out_specs=tokd,
        out_shape=jax.ShapeDtypeStruct((T, D), jnp.float32),
    )(o, x, wo, ln2, rw, wg, wu, wd, fln)


def kernel(input_ids, positions, spec_hidden, emb_table, enorm_w, hnorm_w,
           eh_proj_w, ln1_w, wq, wk, wv, wo, ln2_w, router_w, w_gate, w_up,
           w_down, final_ln_w):
    ids = input_ids.astype(jnp.int32)
    emb = _embed_gather(emb_table, ids)

    inv = 1.0 / (10000.0 ** (jnp.arange(0, DH, 2, dtype=jnp.float32) / DH))
    ang = positions.astype(jnp.float32)[:, None] * inv[None, :]
    cos_t = jnp.tile(jnp.cos(ang), (1, 2 * H))
    sin_t = jnp.tile(jnp.sin(ang), (1, 2 * H))

    def perm(w):
        w4 = w.reshape(D, H, 2, DH // 2)
        return jnp.concatenate([-w4[:, :, 1], w4[:, :, 0]], axis=2).reshape(D, HD)

    res, q, k, v = _prelude(emb, spec_hidden, eh_proj_w, wq, perm(wq),
                            wk, perm(wk), wv, cos_t, sin_t,
                            enorm_w.reshape(1, D), hnorm_w.reshape(1, D),
                            ln1_w.reshape(1, D))
    o = _attention(q, k, v)
    return _post(o, res, wo, ln2_w.reshape(1, D), router_w, w_gate, w_up,
                 w_down, final_ln_w.reshape(1, D))


# speedup vs baseline: 1.3478x; 1.2623x over previous
import functools

import jax
import jax.numpy as jnp
from jax import lax
from jax.experimental import pallas as pl
from jax.experimental.pallas import tpu as pltpu
from jax.experimental.pallas import tpu_sc as plsc

T = 2048
D = 1024
H = 16
DH = 64
E = 8
F = 256
EPS = 1e-6
HD = H * DH

BT = 256
BQ = 512


def _rms(x, w):
    var = jnp.mean(x * x, axis=-1, keepdims=True)
    return x * lax.rsqrt(var + EPS) * w


def _embed_gather(table, idx):
    info = plsc.get_sparse_core_info()
    nw = info.num_cores * info.num_subcores
    b_per_w = T // nw
    mesh = plsc.VectorSubcoreMesh(core_axis_name="c", subcore_axis_name="s")

    @functools.partial(
        pl.kernel,
        mesh=mesh,
        out_type=jax.ShapeDtypeStruct((T, D), jnp.float32),
        scratch_types=[
            pltpu.VMEM((b_per_w,), jnp.int32),
            pltpu.VMEM((b_per_w, D), jnp.float32),
            pltpu.SemaphoreType.DMA,
        ],
    )
    def gather_k(table_hbm, idx_hbm, out_hbm, idx_v, rows_v, sem):
        wid = lax.axis_index("s") * info.num_cores + lax.axis_index("c")
        base = wid * b_per_w
        pltpu.sync_copy(idx_hbm.at[pl.ds(base, b_per_w)], idx_v)
        pltpu.async_copy(table_hbm.at[idx_v], rows_v, sem).wait()
        pltpu.sync_copy(rows_v, out_hbm.at[pl.ds(base, b_per_w)])

    return gather_k(table, idx)


def _rope_t(xt, c, s):
    parts = []
    for h in range(H):
        top = xt[h * DH:h * DH + DH // 2]
        bot = xt[h * DH + DH // 2:(h + 1) * DH]
        parts.append(jnp.concatenate([top * c - bot * s, bot * c + top * s],
                                     axis=0))
    return jnp.concatenate(parts, axis=0)


def _prelude_body(emb_ref, spec_ref, ehp_ref, wq_ref, wk_ref, wv_ref,
                  cos_ref, sin_ref, enw_ref, hnw_ref, ln1_ref,
                  res_ref, q_ref, k_ref, v_ref):
    en = _rms(emb_ref[...], enw_ref[...])
    hn = _rms(spec_ref[...], hnw_ref[...])
    x = (jnp.dot(en, ehp_ref[:D, :], preferred_element_type=jnp.float32)
         + jnp.dot(hn, ehp_ref[D:, :], preferred_element_type=jnp.float32))
    res_ref[...] = x
    hs = _rms(x, ln1_ref[...])
    c, s = cos_ref[...], sin_ref[...]
    tdot = lambda w: lax.dot_general(w, hs, (((0,), (1,)), ((), ())),
                                     preferred_element_type=jnp.float32)
    q_ref[...] = _rope_t(tdot(wq_ref[...]), c, s)
    k_ref[...] = _rope_t(tdot(wk_ref[...]), c, s)
    v_ref[...] = tdot(wv_ref[...])


def _prelude(emb, spec, ehp, wq, wk, wv, cos_t, sin_t, enw, hnw, ln1):
    grid = (T // BT,)
    tok = pl.BlockSpec((BT, D), lambda i: (i, 0))
    tokt = pl.BlockSpec((HD, BT), lambda i: (0, i))
    full = lambda shape: pl.BlockSpec(shape, lambda i: (0,) * len(shape))
    return pl.pallas_call(
        _prelude_body,
        grid=grid,
        in_specs=[tok, tok, full((2 * D, D)), full((D, HD)), full((D, HD)),
                  full((D, HD)),
                  pl.BlockSpec((DH // 2, BT), lambda i: (0, i)),
                  pl.BlockSpec((DH // 2, BT), lambda i: (0, i)),
                  full((1, D)), full((1, D)), full((1, D))],
        out_specs=[tok, tokt, tokt, tokt],
        out_shape=[jax.ShapeDtypeStruct((T, D), jnp.float32),
                   jax.ShapeDtypeStruct((HD, T), jnp.float32),
                   jax.ShapeDtypeStruct((HD, T), jnp.float32),
                   jax.ShapeDtypeStruct((HD, T), jnp.float32)],
    )(emb, spec, ehp, wq, wk, wv, cos_t, sin_t, enw, hnw, ln1)


BK = 512


def _attn_body(q_ref, k_ref, v_ref, o_ref):
    iq = pl.program_id(1)
    rkey = lax.broadcasted_iota(jnp.int32, (BK, BQ), 0)
    cquery = iq * BQ + lax.broadcasted_iota(jnp.int32, (BK, BQ), 1)
    outs = []
    for hh in range(2):
        q = q_ref[hh * DH:(hh + 1) * DH, :]

        def body(j, carry):
            m, l, acc_t = carry
            kc = k_ref[hh * DH:(hh + 1) * DH, pl.ds(j * BK, BK)]
            vc = v_ref[hh * DH:(hh + 1) * DH, pl.ds(j * BK, BK)]
            st = lax.dot_general(kc, q, (((0,), (0,)), ((), ())),
                                 preferred_element_type=jnp.float32) * 0.125
            st = jnp.where(j * BK + rkey <= cquery, st, -1e30)
            m_new = jnp.maximum(m, jnp.max(st, axis=0, keepdims=True))
            alpha = jnp.exp(m - m_new)
            p = jnp.exp(st - m_new)
            l_new = l * alpha + jnp.sum(p, axis=0, keepdims=True)
            acc_t_new = acc_t * alpha + jnp.dot(
                vc, p, preferred_element_type=jnp.float32)
            return m_new, l_new, acc_t_new

        init = (jnp.full((1, BQ), -1e30, jnp.float32),
                jnp.zeros((1, BQ), jnp.float32),
                jnp.zeros((DH, BQ), jnp.float32))
        m, l, acc_t = lax.fori_loop(0, iq + 1, body, init)
        outs.append(acc_t / l)
    o_ref[...] = jnp.concatenate(outs, axis=0)


def _attention(q, k, v):
    grid = (H // 2, T // BQ)
    return pl.pallas_call(
        _attn_body,
        grid=grid,
        in_specs=[
            pl.BlockSpec((2 * DH, BQ), lambda h, i: (h, i)),
            pl.BlockSpec((2 * DH, T), lambda h, i: (h, 0)),
            pl.BlockSpec((2 * DH, T), lambda h, i: (h, 0)),
        ],
        out_specs=pl.BlockSpec((2 * DH, BQ), lambda h, i: (h, i)),
        out_shape=jax.ShapeDtypeStruct((HD, T), jnp.float32),
    )(q, k, v)


def _post_body(o_ref, x_ref, wo_ref, ln2_ref, rw_ref, wg_ref, wu_ref, wd_ref,
               fln_ref, out_ref):
    attn = lax.dot_general(o_ref[...], wo_ref[...], (((0,), (0,)), ((), ())),
                           preferred_element_type=jnp.float32)
    resid = x_ref[...] + attn
    hs = _rms(resid, ln2_ref[...])
    logits = jnp.dot(hs, rw_ref[...], preferred_element_type=jnp.float32)
    m = jnp.max(logits, axis=-1, keepdims=True)
    ex = jnp.exp(logits - m)
    probs = ex / jnp.sum(ex, axis=-1, keepdims=True)
    ii = lax.broadcasted_iota(jnp.int32, (BT, E), 1)
    t1 = jnp.max(probs, axis=-1, keepdims=True)
    a1 = jnp.min(jnp.where(probs == t1, ii, E), axis=-1, keepdims=True)
    sel1 = ii == a1
    p2 = jnp.where(sel1, -1.0, probs)
    t2 = jnp.max(p2, axis=-1, keepdims=True)
    a2 = jnp.min(jnp.where(p2 == t2, ii, E), axis=-1, keepdims=True)
    sel2 = ii == a2
    we = (jnp.where(sel1, t1, 0.0) + jnp.where(sel2, t2, 0.0)) / (t1 + t2)
    acc = jnp.zeros((BT, D), jnp.float32)
    for e in range(E):
        g = jnp.dot(hs, wg_ref[e], preferred_element_type=jnp.float32)
        u = jnp.dot(hs, wu_ref[e], preferred_element_type=jnp.float32)
        act = g / (1.0 + jnp.exp(-g)) * u
        acc = acc + we[:, e:e + 1] * jnp.dot(
            act, wd_ref[e], preferred_element_type=jnp.float32)
    out_ref[...] = _rms(resid + acc, fln_ref[...])


def _post(o, x, wo, ln2, rw, wg, wu, wd, fln):
    grid = (T // BT,)
    tokd = pl.BlockSpec((BT, D), lambda i: (i, 0))
    full = lambda shape: pl.BlockSpec(shape, lambda i: (0,) * len(shape))
    return pl.pallas_call(
        _post_body,
        grid=grid,
        in_specs=[pl.BlockSpec((HD, BT), lambda i: (0, i)), tokd,
                  full((HD, D)), full((1, D)), full((D, E)),
                  full((E, D, F)), full((E, D, F)), full((E, F, D)),
                  full((1, D))],
        out_specs=tokd,
        out_shape=jax.ShapeDtypeStruct((T, D), jnp.float32),
    )(o, x, wo, ln2, rw, wg, wu, wd, fln)


def kernel(input_ids, positions, spec_hidden, emb_table, enorm_w, hnorm_w,
           eh_proj_w, ln1_w, wq, wk, wv, wo, ln2_w, router_w, w_gate, w_up,
           w_down, final_ln_w):
    ids = input_ids.astype(jnp.int32)
    emb = _embed_gather(emb_table, ids)

    inv = 1.0 / (10000.0 ** (jnp.arange(0, DH, 2, dtype=jnp.float32) / DH))
    ang = inv[:, None] * positions.astype(jnp.float32)[None, :]
    cos_t = jnp.cos(ang)
    sin_t = jnp.sin(ang)

    res, q, k, v = _prelude(emb, spec_hidden, eh_proj_w, wq, wk, wv,
                            cos_t, sin_t,
                            enorm_w.reshape(1, D), hnorm_w.reshape(1, D),
                            ln1_w.reshape(1, D))
    o = _attention(q, k, v)
    return _post(o, res, wo, ln2_w.reshape(1, D), router_w, w_gate, w_up,
                 w_down, final_ln_w.reshape(1, D))


# speedup vs baseline: 1.3972x; 1.0367x over previous
import functools

import jax
import jax.numpy as jnp
from jax import lax
from jax.experimental import pallas as pl
from jax.experimental.pallas import tpu as pltpu
from jax.experimental.pallas import tpu_sc as plsc

T = 2048
D = 1024
H = 16
DH = 64
E = 8
F = 256
EPS = 1e-6
HD = H * DH

BT = 256
BQ = 512


def _rms(x, w):
    var = jnp.mean(x * x, axis=-1, keepdims=True)
    return x * lax.rsqrt(var + EPS) * w


def _embed_gather(table, idx):
    info = plsc.get_sparse_core_info()
    nw = info.num_cores * info.num_subcores
    b_per_w = T // nw
    mesh = plsc.VectorSubcoreMesh(core_axis_name="c", subcore_axis_name="s")

    @functools.partial(
        pl.kernel,
        mesh=mesh,
        out_type=jax.ShapeDtypeStruct((T, D), jnp.float32),
        scratch_types=[
            pltpu.VMEM((b_per_w,), jnp.int32),
            pltpu.VMEM((b_per_w, D), jnp.float32),
            pltpu.SemaphoreType.DMA,
        ],
    )
    def gather_k(table_hbm, idx_hbm, out_hbm, idx_v, rows_v, sem):
        wid = lax.axis_index("s") * info.num_cores + lax.axis_index("c")
        base = wid * b_per_w
        pltpu.sync_copy(idx_hbm.at[pl.ds(base, b_per_w)], idx_v)
        pltpu.async_copy(table_hbm.at[idx_v], rows_v, sem).wait()
        pltpu.sync_copy(rows_v, out_hbm.at[pl.ds(base, b_per_w)])

    return gather_k(table, idx)


def _rope_t(xt, c, s):
    parts = []
    for h in range(H):
        top = xt[h * DH:h * DH + DH // 2]
        bot = xt[h * DH + DH // 2:(h + 1) * DH]
        parts.append(jnp.concatenate([top * c - bot * s, bot * c + top * s],
                                     axis=0))
    return jnp.concatenate(parts, axis=0)


def _prelude_body(emb_ref, spec_ref, ehp_ref, wq_ref, wk_ref, wv_ref,
                  cos_ref, sin_ref, enw_ref, hnw_ref, ln1_ref,
                  res_ref, q_ref, k_ref, v_ref):
    en = _rms(emb_ref[...], enw_ref[...])
    hn = _rms(spec_ref[...], hnw_ref[...])
    x = (jnp.dot(en, ehp_ref[:D, :], preferred_element_type=jnp.float32)
         + jnp.dot(hn, ehp_ref[D:, :], preferred_element_type=jnp.float32))
    res_ref[...] = x
    hs = _rms(x, ln1_ref[...])
    c, s = cos_ref[...], sin_ref[...]
    tdot = lambda w: lax.dot_general(w, hs, (((0,), (1,)), ((), ())),
                                     preferred_element_type=jnp.float32)
    q_ref[...] = _rope_t(tdot(wq_ref[...]), c, s)
    k_ref[...] = _rope_t(tdot(wk_ref[...]), c, s)
    v_ref[...] = tdot(wv_ref[...])


def _prelude(emb, spec, ehp, wq, wk, wv, cos_t, sin_t, enw, hnw, ln1):
    grid = (T // BT,)
    tok = pl.BlockSpec((BT, D), lambda i: (i, 0))
    tokt = pl.BlockSpec((HD, BT), lambda i: (0, i))
    full = lambda shape: pl.BlockSpec(shape, lambda i: (0,) * len(shape))
    return pl.pallas_call(
        _prelude_body,
        grid=grid,
        in_specs=[tok, tok, full((2 * D, D)), full((D, HD)), full((D, HD)),
                  full((D, HD)),
                  pl.BlockSpec((DH // 2, BT), lambda i: (0, i)),
                  pl.BlockSpec((DH // 2, BT), lambda i: (0, i)),
                  full((1, D)), full((1, D)), full((1, D))],
        out_specs=[tok, tokt, tokt, tokt],
        out_shape=[jax.ShapeDtypeStruct((T, D), jnp.float32),
                   jax.ShapeDtypeStruct((HD, T), jnp.float32),
                   jax.ShapeDtypeStruct((HD, T), jnp.float32),
                   jax.ShapeDtypeStruct((HD, T), jnp.float32)],
    )(emb, spec, ehp, wq, wk, wv, cos_t, sin_t, enw, hnw, ln1)


BK = 512


def _attn_body(q_ref, k_ref, v_ref, o_ref):
    iq = pl.program_id(1)
    dmask = (lax.broadcasted_iota(jnp.int32, (BK, BQ), 0)
             <= lax.broadcasted_iota(jnp.int32, (BK, BQ), 1))
    outs = []
    for hh in range(2):
        q = q_ref[hh * DH:(hh + 1) * DH, :]

        def chunk(j, carry, masked):
            m, l, acc_t = carry
            kc = k_ref[hh * DH:(hh + 1) * DH, pl.ds(j * BK, BK)]
            vc = v_ref[hh * DH:(hh + 1) * DH, pl.ds(j * BK, BK)]
            st = lax.dot_general(kc, q, (((0,), (0,)), ((), ())),
                                 preferred_element_type=jnp.float32) * 0.125
            if masked:
                st = jnp.where(dmask, st, -1e30)
            m_new = jnp.maximum(m, jnp.max(st, axis=0, keepdims=True))
            alpha = jnp.exp(m - m_new)
            p = jnp.exp(st - m_new)
            l_new = l * alpha + jnp.sum(p, axis=0, keepdims=True)
            acc_t_new = acc_t * alpha + jnp.dot(
                vc, p, preferred_element_type=jnp.float32)
            return m_new, l_new, acc_t_new

        init = (jnp.full((1, BQ), -1e30, jnp.float32),
                jnp.zeros((1, BQ), jnp.float32),
                jnp.zeros((DH, BQ), jnp.float32))
        carry = lax.fori_loop(0, iq, lambda j, c: chunk(j, c, False), init)
        m, l, acc_t = chunk(iq, carry, True)
        outs.append(acc_t / l)
    o_ref[...] = jnp.concatenate(outs, axis=0)


def _attention(q, k, v):
    grid = (H // 2, T // BQ)
    return pl.pallas_call(
        _attn_body,
        grid=grid,
        in_specs=[
            pl.BlockSpec((2 * DH, BQ), lambda h, i: (h, i)),
            pl.BlockSpec((2 * DH, T), lambda h, i: (h, 0)),
            pl.BlockSpec((2 * DH, T), lambda h, i: (h, 0)),
        ],
        out_specs=pl.BlockSpec((2 * DH, BQ), lambda h, i: (h, i)),
        out_shape=jax.ShapeDtypeStruct((HD, T), jnp.float32),
    )(q, k, v)


def _post_body(o_ref, x_ref, wo_ref, ln2_ref, rw_ref, wg_ref, wu_ref, wd_ref,
               fln_ref, out_ref):
    attn = lax.dot_general(o_ref[...], wo_ref[...], (((0,), (0,)), ((), ())),
                           preferred_element_type=jnp.float32)
    resid = x_ref[...] + attn
    hs = _rms(resid, ln2_ref[...])
    logits = jnp.dot(hs, rw_ref[...], preferred_element_type=jnp.float32)
    m = jnp.max(logits, axis=-1, keepdims=True)
    ex = jnp.exp(logits - m)
    probs = ex / jnp.sum(ex, axis=-1, keepdims=True)
    ii = lax.broadcasted_iota(jnp.int32, (BT, E), 1)
    t1 = jnp.max(probs, axis=-1, keepdims=True)
    a1 = jnp.min(jnp.where(probs == t1, ii, E), axis=-1, keepdims=True)
    sel1 = ii == a1
    p2 = jnp.where(sel1, -1.0, probs)
    t2 = jnp.max(p2, axis=-1, keepdims=True)
    a2 = jnp.min(jnp.where(p2 == t2, ii, E), axis=-1, keepdims=True)
    sel2 = ii == a2
    we = (jnp.where(sel1, t1, 0.0) + jnp.where(sel2, t2, 0.0)) / (t1 + t2)
    acc = jnp.zeros((BT, D), jnp.float32)
    for e in range(E):
        g = jnp.dot(hs, wg_ref[e], preferred_element_type=jnp.float32)
        u = jnp.dot(hs, wu_ref[e], preferred_element_type=jnp.float32)
        act = g / (1.0 + jnp.exp(-g)) * u
        acc = acc + we[:, e:e + 1] * jnp.dot(
            act, wd_ref[e], preferred_element_type=jnp.float32)
    out_ref[...] = _rms(resid + acc, fln_ref[...])


def _post(o, x, wo, ln2, rw, wg, wu, wd, fln):
    grid = (T // BT,)
    tokd = pl.BlockSpec((BT, D), lambda i: (i, 0))
    full = lambda shape: pl.BlockSpec(shape, lambda i: (0,) * len(shape))
    return pl.pallas_call(
        _post_body,
        grid=grid,
        in_specs=[pl.BlockSpec((HD, BT), lambda i: (0, i)), tokd,
                  full((HD, D)), full((1, D)), full((D, E)),
                  full((E, D, F)), full((E, D, F)), full((E, F, D)),
                  full((1, D))],
        out_specs=tokd,
        out_shape=jax.ShapeDtypeStruct((T, D), jnp.float32),
    )(o, x, wo, ln2, rw, wg, wu, wd, fln)


def kernel(input_ids, positions, spec_hidden, emb_table, enorm_w, hnorm_w,
           eh_proj_w, ln1_w, wq, wk, wv, wo, ln2_w, router_w, w_gate, w_up,
           w_down, final_ln_w):
    ids = input_ids.astype(jnp.int32)
    emb = _embed_gather(emb_table, ids)

    inv = 1.0 / (10000.0 ** (jnp.arange(0, DH, 2, dtype=jnp.float32) / DH))
    ang = inv[:, None] * positions.astype(jnp.float32)[None, :]
    cos_t = jnp.cos(ang)
    sin_t = jnp.sin(ang)

    res, q, k, v = _prelude(emb, spec_hidden, eh_proj_w, wq, wk, wv,
                            cos_t, sin_t,
                            enorm_w.reshape(1, D), hnorm_w.reshape(1, D),
                            ln1_w.reshape(1, D))
    o = _attention(q, k, v)
    return _post(o, res, wo, ln2_w.reshape(1, D), router_w, w_gate, w_up,
                 w_down, final_ln_w.reshape(1, D))


# speedup vs baseline: 1.4185x; 1.0152x over previous
import functools

import jax
import jax.numpy as jnp
from jax import lax
from jax.experimental import pallas as pl
from jax.experimental.pallas import tpu as pltpu
from jax.experimental.pallas import tpu_sc as plsc

T = 2048
D = 1024
H = 16
DH = 64
E = 8
F = 256
EPS = 1e-6
HD = H * DH

BT = 256
BQ = 512


def _rms(x, w):
    var = jnp.mean(x * x, axis=-1, keepdims=True)
    return x * lax.rsqrt(var + EPS) * w


def _embed_gather(table, idx):
    info = plsc.get_sparse_core_info()
    nw = info.num_cores * info.num_subcores
    b_per_w = T // nw
    mesh = plsc.VectorSubcoreMesh(core_axis_name="c", subcore_axis_name="s")

    @functools.partial(
        pl.kernel,
        mesh=mesh,
        out_type=jax.ShapeDtypeStruct((T, D), jnp.float32),
        scratch_types=[
            pltpu.VMEM((b_per_w,), jnp.int32),
            pltpu.VMEM((b_per_w, D), jnp.float32),
            pltpu.SemaphoreType.DMA,
        ],
    )
    def gather_k(table_hbm, idx_hbm, out_hbm, idx_v, rows_v, sem):
        wid = lax.axis_index("s") * info.num_cores + lax.axis_index("c")
        base = wid * b_per_w
        pltpu.sync_copy(idx_hbm.at[pl.ds(base, b_per_w)], idx_v)
        pltpu.async_copy(table_hbm.at[idx_v], rows_v, sem).wait()
        pltpu.sync_copy(rows_v, out_hbm.at[pl.ds(base, b_per_w)])

    return gather_k(table, idx)


def _rope_t(xt, c, s):
    parts = []
    for h in range(H):
        top = xt[h * DH:h * DH + DH // 2]
        bot = xt[h * DH + DH // 2:(h + 1) * DH]
        parts.append(jnp.concatenate([top * c - bot * s, bot * c + top * s],
                                     axis=0))
    return jnp.concatenate(parts, axis=0)


def _prelude_body(emb_ref, spec_ref, ehp_ref, wq_ref, wk_ref, wv_ref,
                  cos_ref, sin_ref, enw_ref, hnw_ref, ln1_ref,
                  res_ref, q_ref, k_ref, v_ref):
    en = _rms(emb_ref[...], enw_ref[...])
    hn = _rms(spec_ref[...], hnw_ref[...])
    x = (jnp.dot(en, ehp_ref[:D, :], preferred_element_type=jnp.float32)
         + jnp.dot(hn, ehp_ref[D:, :], preferred_element_type=jnp.float32))
    res_ref[...] = x
    hs = _rms(x, ln1_ref[...])
    c, s = cos_ref[...], sin_ref[...]
    tdot = lambda w: lax.dot_general(w, hs, (((0,), (1,)), ((), ())),
                                     preferred_element_type=jnp.float32)
    q_ref[...] = _rope_t(tdot(wq_ref[...]), c, s)
    k_ref[...] = _rope_t(tdot(wk_ref[...]), c, s)
    v_ref[...] = tdot(wv_ref[...])


def _prelude(emb, spec, ehp, wq, wk, wv, cos_t, sin_t, enw, hnw, ln1):
    grid = (T // BT,)
    tok = pl.BlockSpec((BT, D), lambda i: (i, 0))
    tokt = pl.BlockSpec((HD, BT), lambda i: (0, i))
    full = lambda shape: pl.BlockSpec(shape, lambda i: (0,) * len(shape))
    return pl.pallas_call(
        _prelude_body,
        grid=grid,
        in_specs=[tok, tok, full((2 * D, D)), full((D, HD)), full((D, HD)),
                  full((D, HD)),
                  pl.BlockSpec((DH // 2, BT), lambda i: (0, i)),
                  pl.BlockSpec((DH // 2, BT), lambda i: (0, i)),
                  full((1, D)), full((1, D)), full((1, D))],
        out_specs=[tok, tokt, tokt, tokt],
        out_shape=[jax.ShapeDtypeStruct((T, D), jnp.float32),
                   jax.ShapeDtypeStruct((HD, T), jnp.float32),
                   jax.ShapeDtypeStruct((HD, T), jnp.float32),
                   jax.ShapeDtypeStruct((HD, T), jnp.float32)],
    )(emb, spec, ehp, wq, wk, wv, cos_t, sin_t, enw, hnw, ln1)


BK = 512


def _attn_body(q_ref, k_ref, v_ref, o_ref):
    iq = pl.program_id(1)
    dmask = (lax.broadcasted_iota(jnp.int32, (BK, BQ), 0)
             <= lax.broadcasted_iota(jnp.int32, (BK, BQ), 1))
    outs = []
    for hh in range(2):
        q = q_ref[hh * DH:(hh + 1) * DH, :]
        ones_blk = jnp.ones((8, BK), jnp.float32)

        def chunk(j, carry, masked):
            m, acc_t = carry
            kc = k_ref[hh * DH:(hh + 1) * DH, pl.ds(j * BK, BK)]
            vc = jnp.concatenate(
                [v_ref[hh * DH:(hh + 1) * DH, pl.ds(j * BK, BK)], ones_blk],
                axis=0)
            st = lax.dot_general(kc, q, (((0,), (0,)), ((), ())),
                                 preferred_element_type=jnp.float32) * 0.125
            if masked:
                st = jnp.where(dmask, st, -1e30)
            m_new = jnp.maximum(m, jnp.max(st, axis=0, keepdims=True))
            alpha = jnp.exp(m - m_new)
            p = jnp.exp(st - m_new)
            acc_t_new = acc_t * alpha + jnp.dot(
                vc, p, preferred_element_type=jnp.float32)
            return m_new, acc_t_new

        init = (jnp.full((1, BQ), -1e30, jnp.float32),
                jnp.zeros((DH + 8, BQ), jnp.float32))
        carry = lax.fori_loop(0, iq, lambda j, c: chunk(j, c, False), init)
        m, acc_t = chunk(iq, carry, True)
        outs.append(acc_t[:DH] / acc_t[DH:DH + 1])
    o_ref[...] = jnp.concatenate(outs, axis=0)


def _attention(q, k, v):
    grid = (H // 2, T // BQ)
    return pl.pallas_call(
        _attn_body,
        grid=grid,
        in_specs=[
            pl.BlockSpec((2 * DH, BQ), lambda h, i: (h, i)),
            pl.BlockSpec((2 * DH, T), lambda h, i: (h, 0)),
            pl.BlockSpec((2 * DH, T), lambda h, i: (h, 0)),
        ],
        out_specs=pl.BlockSpec((2 * DH, BQ), lambda h, i: (h, i)),
        out_shape=jax.ShapeDtypeStruct((HD, T), jnp.float32),
    )(q, k, v)


def _post_body(o_ref, x_ref, wo_ref, ln2_ref, rw_ref, wg_ref, wu_ref, wd_ref,
               fln_ref, out_ref):
    attn = lax.dot_general(o_ref[...], wo_ref[...], (((0,), (0,)), ((), ())),
                           preferred_element_type=jnp.float32)
    resid = x_ref[...] + attn
    hs = _rms(resid, ln2_ref[...])
    logits = jnp.dot(hs, rw_ref[...], preferred_element_type=jnp.float32)
    m = jnp.max(logits, axis=-1, keepdims=True)
    ex = jnp.exp(logits - m)
    probs = ex / jnp.sum(ex, axis=-1, keepdims=True)
    ii = lax.broadcasted_iota(jnp.int32, (BT, E), 1)
    t1 = jnp.max(probs, axis=-1, keepdims=True)
    a1 = jnp.min(jnp.where(probs == t1, ii, E), axis=-1, keepdims=True)
    sel1 = ii == a1
    p2 = jnp.where(sel1, -1.0, probs)
    t2 = jnp.max(p2, axis=-1, keepdims=True)
    a2 = jnp.min(jnp.where(p2 == t2, ii, E), axis=-1, keepdims=True)
    sel2 = ii == a2
    we = (jnp.where(sel1, t1, 0.0) + jnp.where(sel2, t2, 0.0)) / (t1 + t2)
    acc = jnp.zeros((BT, D), jnp.float32)
    for e in range(E):
        g = jnp.dot(hs, wg_ref[e], preferred_element_type=jnp.float32)
        u = jnp.dot(hs, wu_ref[e], preferred_element_type=jnp.float32)
        act = g / (1.0 + jnp.exp(-g)) * u
        acc = acc + we[:, e:e + 1] * jnp.dot(
            act, wd_ref[e], preferred_element_type=jnp.float32)
    out_ref[...] = _rms(resid + acc, fln_ref[...])


def _post(o, x, wo, ln2, rw, wg, wu, wd, fln):
    grid = (T // BT,)
    tokd = pl.BlockSpec((BT, D), lambda i: (i, 0))
    full = lambda shape: pl.BlockSpec(shape, lambda i: (0,) * len(shape))
    return pl.pallas_call(
        _post_body,
        grid=grid,
        in_specs=[pl.BlockSpec((HD, BT), lambda i: (0, i)), tokd,
                  full((HD, D)), full((1, D)), full((D, E)),
                  full((E, D, F)), full((E, D, F)), full((E, F, D)),
                  full((1, D))],
        out_specs=tokd,
        out_shape=jax.ShapeDtypeStruct((T, D), jnp.float32),
    )(o, x, wo, ln2, rw, wg, wu, wd, fln)


def kernel(input_ids, positions, spec_hidden, emb_table, enorm_w, hnorm_w,
           eh_proj_w, ln1_w, wq, wk, wv, wo, ln2_w, router_w, w_gate, w_up,
           w_down, final_ln_w):
    ids = input_ids.astype(jnp.int32)
    emb = _embed_gather(emb_table, ids)

    inv = 1.0 / (10000.0 ** (jnp.arange(0, DH, 2, dtype=jnp.float32) / DH))
    ang = inv[:, None] * positions.astype(jnp.float32)[None, :]
    cos_t = jnp.cos(ang)
    sin_t = jnp.sin(ang)

    res, q, k, v = _prelude(emb, spec_hidden, eh_proj_w, wq, wk, wv,
                            cos_t, sin_t,
                            enorm_w.reshape(1, D), hnorm_w.reshape(1, D),
                            ln1_w.reshape(1, D))
    o = _attention(q, k, v)
    return _post(o, res, wo, ln2_w.reshape(1, D), router_w, w_gate, w_up,
                 w_down, final_ln_w.reshape(1, D))


# speedup vs baseline: 1.4526x; 1.0240x over previous
import functools

import jax
import jax.numpy as jnp
from jax import lax
from jax.experimental import pallas as pl
from jax.experimental.pallas import tpu as pltpu
from jax.experimental.pallas import tpu_sc as plsc

T = 2048
D = 1024
H = 16
DH = 64
E = 8
F = 256
EPS = 1e-6
HD = H * DH

BT = 256
BQ = 512


def _rms(x, w):
    var = jnp.mean(x * x, axis=-1, keepdims=True)
    return x * lax.rsqrt(var + EPS) * w


def _embed_gather(table, idx):
    info = plsc.get_sparse_core_info()
    nw = info.num_cores * info.num_subcores
    b_per_w = T // nw
    mesh = plsc.VectorSubcoreMesh(core_axis_name="c", subcore_axis_name="s")

    @functools.partial(
        pl.kernel,
        mesh=mesh,
        out_type=jax.ShapeDtypeStruct((T, D), jnp.float32),
        scratch_types=[
            pltpu.VMEM((b_per_w,), jnp.int32),
            pltpu.VMEM((b_per_w, D), jnp.float32),
            pltpu.SemaphoreType.DMA,
        ],
    )
    def gather_k(table_hbm, idx_hbm, out_hbm, idx_v, rows_v, sem):
        wid = lax.axis_index("s") * info.num_cores + lax.axis_index("c")
        base = wid * b_per_w
        pltpu.sync_copy(idx_hbm.at[pl.ds(base, b_per_w)], idx_v)
        pltpu.async_copy(table_hbm.at[idx_v], rows_v, sem).wait()
        pltpu.sync_copy(rows_v, out_hbm.at[pl.ds(base, b_per_w)])

    return gather_k(table, idx)


def _rope_t(xt, c, s):
    parts = []
    for h in range(H):
        top = xt[h * DH:h * DH + DH // 2]
        bot = xt[h * DH + DH // 2:(h + 1) * DH]
        parts.append(jnp.concatenate([top * c - bot * s, bot * c + top * s],
                                     axis=0))
    return jnp.concatenate(parts, axis=0)


def _prelude_body(emb_ref, spec_ref, ehp_ref, wq_ref, wk_ref, wv_ref,
                  cos_ref, sin_ref, enw_ref, hnw_ref, ln1_ref,
                  res_ref, q_ref, k_ref, v_ref):
    en = _rms(emb_ref[...], enw_ref[...])
    hn = _rms(spec_ref[...], hnw_ref[...])
    x = (jnp.dot(en, ehp_ref[:D, :], preferred_element_type=jnp.float32)
         + jnp.dot(hn, ehp_ref[D:, :], preferred_element_type=jnp.float32))
    res_ref[...] = x
    hs = _rms(x, ln1_ref[...])
    c, s = cos_ref[...], sin_ref[...]
    tdot = lambda w: lax.dot_general(w, hs, (((0,), (1,)), ((), ())),
                                     preferred_element_type=jnp.float32)
    q_ref[...] = _rope_t(tdot(wq_ref[...]), c, s).astype(jnp.bfloat16)
    k_ref[...] = _rope_t(tdot(wk_ref[...]), c, s).astype(jnp.bfloat16)
    v_ref[...] = tdot(wv_ref[...]).astype(jnp.bfloat16)


def _prelude(emb, spec, ehp, wq, wk, wv, cos_t, sin_t, enw, hnw, ln1):
    grid = (T // BT,)
    tok = pl.BlockSpec((BT, D), lambda i: (i, 0))
    tokt = pl.BlockSpec((HD, BT), lambda i: (0, i))
    full = lambda shape: pl.BlockSpec(shape, lambda i: (0,) * len(shape))
    return pl.pallas_call(
        _prelude_body,
        grid=grid,
        in_specs=[tok, tok, full((2 * D, D)), full((D, HD)), full((D, HD)),
                  full((D, HD)),
                  pl.BlockSpec((DH // 2, BT), lambda i: (0, i)),
                  pl.BlockSpec((DH // 2, BT), lambda i: (0, i)),
                  full((1, D)), full((1, D)), full((1, D))],
        out_specs=[tok, tokt, tokt, tokt],
        out_shape=[jax.ShapeDtypeStruct((T, D), jnp.float32),
                   jax.ShapeDtypeStruct((HD, T), jnp.bfloat16),
                   jax.ShapeDtypeStruct((HD, T), jnp.bfloat16),
                   jax.ShapeDtypeStruct((HD, T), jnp.bfloat16)],
    )(emb, spec, ehp, wq, wk, wv, cos_t, sin_t, enw, hnw, ln1)


BK = 512


def _attn_post_body(q_ref, k_ref, v_ref, x_ref, wo_ref, ln2_ref, rw_ref,
                    wg_ref, wu_ref, wd_ref, fln_ref, out_ref, o_sc):
    i = pl.program_id(0)
    h = pl.program_id(1)
    dmask = (lax.broadcasted_iota(jnp.int32, (BK, BQ), 0)
             <= lax.broadcasted_iota(jnp.int32, (BK, BQ), 1))
    outs = []
    for hh in range(2):
        q = q_ref[0, hh * DH:(hh + 1) * DH, :]
        ones_blk = jnp.ones((8, BK), jnp.bfloat16)

        def chunk(j, carry, masked):
            m, acc_t = carry
            kc = k_ref[h, hh * DH:(hh + 1) * DH, pl.ds(j * BK, BK)]
            vc = jnp.concatenate(
                [v_ref[h, hh * DH:(hh + 1) * DH, pl.ds(j * BK, BK)], ones_blk],
                axis=0)
            st = lax.dot_general(kc, q, (((0,), (0,)), ((), ())),
                                 preferred_element_type=jnp.float32) * 0.125
            if masked:
                st = jnp.where(dmask, st, -1e30)
            m_new = jnp.maximum(m, jnp.max(st, axis=0, keepdims=True))
            alpha = jnp.exp(m - m_new)
            p = jnp.exp(st - m_new)
            acc_t_new = acc_t * alpha + jnp.dot(
                vc, p.astype(jnp.bfloat16), preferred_element_type=jnp.float32)
            return m_new, acc_t_new

        init = (jnp.full((1, BQ), -1e30, jnp.float32),
                jnp.zeros((DH + 8, BQ), jnp.float32))
        carry = lax.fori_loop(0, i, lambda j, c: chunk(j, c, False), init)
        m, acc_t = chunk(i, carry, True)
        outs.append(acc_t[:DH] / acc_t[DH:DH + 1])
    o_sc[h] = jnp.concatenate(outs, axis=0)

    @pl.when(h == H // 2 - 1)
    def _post_stage():
        o_all = o_sc[...].reshape(HD, BQ)
        attn = lax.dot_general(o_all, wo_ref[...], (((0,), (0,)), ((), ())),
                               preferred_element_type=jnp.float32)
        resid = x_ref[...] + attn
        hs = _rms(resid, ln2_ref[...])
        hsb = hs.astype(jnp.bfloat16)
        logits = jnp.dot(hs, rw_ref[...], preferred_element_type=jnp.float32)
        mx = jnp.max(logits, axis=-1, keepdims=True)
        ex = jnp.exp(logits - mx)
        probs = ex / jnp.sum(ex, axis=-1, keepdims=True)
        ii = lax.broadcasted_iota(jnp.int32, (BQ, E), 1)
        t1 = jnp.max(probs, axis=-1, keepdims=True)
        a1 = jnp.min(jnp.where(probs == t1, ii, E), axis=-1, keepdims=True)
        sel1 = ii == a1
        p2 = jnp.where(sel1, -1.0, probs)
        t2 = jnp.max(p2, axis=-1, keepdims=True)
        a2 = jnp.min(jnp.where(p2 == t2, ii, E), axis=-1, keepdims=True)
        sel2 = ii == a2
        we = (jnp.where(sel1, t1, 0.0) + jnp.where(sel2, t2, 0.0)) / (t1 + t2)
        acc = jnp.zeros((BQ, D), jnp.float32)
        for e in range(E):
            g = jnp.dot(hsb, wg_ref[e], preferred_element_type=jnp.float32)
            u = jnp.dot(hsb, wu_ref[e], preferred_element_type=jnp.float32)
            act = (g / (1.0 + jnp.exp(-g)) * u).astype(jnp.bfloat16)
            acc = acc + we[:, e:e + 1] * jnp.dot(
                act, wd_ref[e], preferred_element_type=jnp.float32)
        out_ref[...] = _rms(resid + acc, fln_ref[...])


def _attn_post(q, k, v, x, wo, ln2, rw, wg, wu, wd, fln):
    grid = (T // BQ, H // 2)
    full = lambda shape: pl.BlockSpec(shape, lambda i, h: (0,) * len(shape))
    return pl.pallas_call(
        _attn_post_body,
        grid=grid,
        in_specs=[
            pl.BlockSpec((1, 2 * DH, BQ), lambda i, h: (h, 0, i)),
            full((H // 2, 2 * DH, T)),
            full((H // 2, 2 * DH, T)),
            pl.BlockSpec((BQ, D), lambda i, h: (i, 0)),
            full((HD, D)), full((1, D)), full((D, E)),
            full((E, D, F)), full((E, D, F)), full((E, F, D)),
            full((1, D)),
        ],
        out_specs=pl.BlockSpec((BQ, D), lambda i, h: (i, 0)),
        out_shape=jax.ShapeDtypeStruct((T, D), jnp.float32),
        scratch_shapes=[pltpu.VMEM((H // 2, 2 * DH, BQ), jnp.float32)],
    )(q, k, v, x, wo, ln2, rw, wg, wu, wd, fln)


def kernel(input_ids, positions, spec_hidden, emb_table, enorm_w, hnorm_w,
           eh_proj_w, ln1_w, wq, wk, wv, wo, ln2_w, router_w, w_gate, w_up,
           w_down, final_ln_w):
    ids = input_ids.astype(jnp.int32)
    emb = _embed_gather(emb_table, ids)

    inv = 1.0 / (10000.0 ** (jnp.arange(0, DH, 2, dtype=jnp.float32) / DH))
    ang = inv[:, None] * positions.astype(jnp.float32)[None, :]
    cos_t = jnp.cos(ang)
    sin_t = jnp.sin(ang)

    res, q, k, v = _prelude(emb, spec_hidden, eh_proj_w, wq, wk, wv,
                            cos_t, sin_t,
                            enorm_w.reshape(1, D), hnorm_w.reshape(1, D),
                            ln1_w.reshape(1, D))
    q3 = q.reshape(H // 2, 2 * DH, T)
    k3 = k.reshape(H // 2, 2 * DH, T)
    v3 = v.reshape(H // 2, 2 * DH, T)
    bf = lambda w: w.astype(jnp.bfloat16)
    return _attn_post(q3, k3, v3, res, wo, ln2_w.reshape(1, D), router_w,
                      bf(w_gate), bf(w_up), bf(w_down),
                      final_ln_w.reshape(1, D))
